# baseline (speedup 1.0000x reference)
import functools

import jax
import jax.numpy as jnp
from jax import lax
from jax.experimental import pallas as pl
from jax.experimental.pallas import tpu as pltpu
from jax.experimental.pallas import tpu_sc as plsc

N = 262144
DIM = 128
L = 16
NC = 2
NS = 16
NW = NC * NS
ROWS_PER_W = N // NW
C = 128
NCHUNK = ROWS_PER_W // C

_mesh = plsc.VectorSubcoreMesh(core_axis_name="c", subcore_axis_name="s")


@functools.partial(
    pl.kernel,
    out_type=jax.ShapeDtypeStruct((N * DIM,), jnp.float32),
    mesh=_mesh,
    scratch_types=[
        pltpu.VMEM((DIM,), jnp.int32),
        pltpu.VMEM((C * DIM,), jnp.float32),
        pltpu.VMEM((C * DIM,), jnp.float32),
    ],
)
def _sc_permute(x_hbm, perm_hbm, out_hbm, perm_v, inb, outb):
    wid = lax.axis_index("s") * NC + lax.axis_index("c")
    base = wid * (ROWS_PER_W * DIM)
    pltpu.sync_copy(perm_hbm, perm_v)
    cidx = [perm_v[pl.ds(j * L, L)] for j in range(DIM // L)]

    def chunk(i, _):
        off = base + i * (C * DIM)
        pltpu.sync_copy(x_hbm.at[pl.ds(off, C * DIM)], inb)

        def row(r, _):
            rbase = r * DIM
            for j in range(DIM // L):
                v = inb[pl.ds(rbase + (DIM // L - 1 - j) * L, L)]
                outb[pl.ds(rbase + j * L, L)] = jnp.flip(v, 0)
            return 0

        lax.fori_loop(0, C, row, 0)
        pltpu.sync_copy(outb, out_hbm.at[pl.ds(off, C * DIM)])
        return 0

    lax.fori_loop(0, NCHUNK, chunk, 0)


def kernel(x, perm):
    out = _sc_permute(x.reshape(-1), perm)
    return out.reshape(N, DIM)
